# baseline (speedup 1.0000x reference)
import jax
import jax.numpy as jnp
from jax import lax
from jax.experimental import pallas as pl
from jax.experimental.pallas import tpu as pltpu

_SORT8 = [(0, 1), (2, 3), (4, 5), (6, 7),
          (0, 2), (1, 3), (4, 6), (5, 7),
          (1, 2), (5, 6),
          (0, 4), (1, 5), (2, 6), (3, 7),
          (2, 4), (3, 5),
          (1, 2), (3, 4), (5, 6)]

_BITONIC8 = [(0, 4), (1, 5), (2, 6), (3, 7),
             (0, 2), (1, 3), (4, 6), (5, 7),
             (0, 1), (2, 3), (4, 5), (6, 7)]

_K_VALS = (16, 32)


def _apply_network(planes, network):
    p = list(planes)
    for i, j in network:
        hi = jnp.maximum(p[i], p[j])
        lo = jnp.minimum(p[i], p[j])
        p[i], p[j] = hi, lo
    return p


def _merge_top8(a, b):
    c = [jnp.maximum(a[i], b[7 - i]) for i in range(8)]
    d = jnp.minimum(a[0], b[7])
    for i in range(1, 8):
        d = jnp.maximum(d, jnp.minimum(a[i], b[7 - i]))
    return _apply_network(c, _BITONIC8), d


def _extract_thresholds(mat, k_values):
    thresholds = {}
    k_max = max(k_values)
    w = mat
    for i in range(k_max):
        m = jnp.max(w, axis=1, keepdims=True)
        if (i + 1) in k_values:
            thresholds[i + 1] = m
        if i + 1 < k_max:
            w = jnp.where(w >= m, -1.0, w)
    return thresholds


def _encode(x_ref, we_ref, be_ref):
    e = jnp.dot(x_ref[...], we_ref[...], preferred_element_type=jnp.float32)
    return jnp.maximum(e + be_ref[...], 0.0)


def _select(e):
    ka, kb = _K_VALS
    br, d = e.shape
    nseg = d // 128
    t16_list, t32_list = [], []
    bad_any = None
    for r in range(0, br, 8):
        er = e[r:r + 8, :]
        planes = [er[:, 128 * j:128 * (j + 1)] for j in range(nseg)]
        sorted_chunks = [_apply_network(planes[8 * g:8 * (g + 1)], _SORT8)
                         for g in range(nseg // 8)]
        top8 = sorted_chunks[0]
        discard_bound = None
        for chunk in sorted_chunks[1:]:
            top8, dmax = _merge_top8(top8, chunk)
            discard_bound = dmax if discard_bound is None else \
                jnp.maximum(discard_bound, dmax)
        reduced = jnp.concatenate(top8, axis=1)

        th = _extract_thresholds(reduced, _K_VALS)
        t16_list.append(th[ka])
        t32_list.append(th[kb])

        if discard_bound is not None:
            db = jnp.max(discard_bound, axis=1, keepdims=True)
            ta, tb = th[ka], th[kb]
            bad_row = ((tb > 0.0) & (db >= tb)) | \
                      ((tb <= 0.0) & (ta > 0.0) & (db >= ta))
            any_r = jnp.max(bad_row.astype(jnp.float32))
            bad_any = any_r if bad_any is None else \
                jnp.maximum(bad_any, any_r)
    if bad_any is None:
        bad_any = jnp.float32(0.0)
    t16 = jnp.concatenate(t16_list, axis=0)
    t32 = jnp.concatenate(t32_list, axis=0)
    return t16, t32, bad_any


def _fast_body(x_ref, we_ref, wd_ref, be_ref, bd_ref,
               f16_ref, f32_ref, r16_ref, r32_ref, flag_ref,
               fb16_s, fb32_s):
    i = pl.program_id(0)
    nt = pl.num_programs(0) - 1

    @pl.when(i > 0)
    def _decode_prev():
        wd = wd_ref[...]
        bd = bd_ref[...]
        r16_ref[...] = jnp.dot(fb16_s[...], wd,
                               preferred_element_type=jnp.float32) + bd
        r32_ref[...] = jnp.dot(fb32_s[...], wd,
                               preferred_element_type=jnp.float32) + bd

    @pl.when(i < nt)
    def _encode_select():
        e = _encode(x_ref, we_ref, be_ref)
        t16, t32, bad_any = _select(e)
        flag_ref[...] = jnp.broadcast_to(bad_any, flag_ref.shape)
        f16 = jnp.where(e >= t16, e, 0.0)
        f32_ = jnp.where(e >= t32, e, 0.0)
        f16_ref[...] = f16
        f32_ref[...] = f32_
        fb16_s[...] = f16.astype(jnp.bfloat16)
        fb32_s[...] = f32_.astype(jnp.bfloat16)


def _exact_body(x_ref, we_ref, wd_ref, be_ref, bd_ref,
                f16_ref, f32_ref, r16_ref, r32_ref):
    ka, kb = _K_VALS
    e = _encode(x_ref, we_ref, be_ref)
    th = _extract_thresholds(e, _K_VALS)
    f16 = jnp.where(e >= th[ka], e, 0.0)
    f32_ = jnp.where(e >= th[kb], e, 0.0)
    f16_ref[...] = f16
    f32_ref[...] = f32_
    wd = wd_ref[...]
    bd = bd_ref[...]
    r16_ref[...] = jnp.dot(f16.astype(jnp.bfloat16), wd,
                           preferred_element_type=jnp.float32) + bd
    r32_ref[...] = jnp.dot(f32_.astype(jnp.bfloat16), wd,
                           preferred_element_type=jnp.float32) + bd


def _in_specs(br, C, D, nt):
    return [
        pl.BlockSpec((br, C), lambda i: (jnp.minimum(i, nt - 1), 0)),
        pl.BlockSpec((C, D), lambda i: (0, 0)),
        pl.BlockSpec((D, C), lambda i: (0, 0)),
        pl.BlockSpec((1, D), lambda i: (0, 0)),
        pl.BlockSpec((1, C), lambda i: (0, 0)),
    ]


def _make_fast_call(br, C, D, N):
    nt = N // br
    cur = lambda i: (jnp.minimum(i, nt - 1), 0)
    prev = lambda i: (jnp.maximum(i - 1, 0), 0)
    return pl.pallas_call(
        _fast_body,
        grid=(nt + 1,),
        in_specs=_in_specs(br, C, D, nt),
        out_specs=[
            pl.BlockSpec((br, D), cur),
            pl.BlockSpec((br, D), cur),
            pl.BlockSpec((br, C), prev),
            pl.BlockSpec((br, C), prev),
            pl.BlockSpec((1, 1, 128),
                         lambda i: (jnp.minimum(i, nt - 1), 0, 0)),
        ],
        out_shape=[
            jax.ShapeDtypeStruct((N, D), jnp.float32),
            jax.ShapeDtypeStruct((N, D), jnp.float32),
            jax.ShapeDtypeStruct((N, C), jnp.float32),
            jax.ShapeDtypeStruct((N, C), jnp.float32),
            jax.ShapeDtypeStruct((nt, 1, 128), jnp.float32),
        ],
        scratch_shapes=[
            pltpu.VMEM((br, D), jnp.bfloat16),
            pltpu.VMEM((br, D), jnp.bfloat16),
        ],
    )


def _make_exact_call(br, C, D, N):
    nt = N // br
    cur = lambda i: (i, 0)
    return pl.pallas_call(
        _exact_body,
        grid=(nt,),
        in_specs=_in_specs(br, C, D, nt + 1),
        out_specs=[
            pl.BlockSpec((br, D), cur),
            pl.BlockSpec((br, D), cur),
            pl.BlockSpec((br, C), cur),
            pl.BlockSpec((br, C), cur),
        ],
        out_shape=[
            jax.ShapeDtypeStruct((N, D), jnp.float32),
            jax.ShapeDtypeStruct((N, D), jnp.float32),
            jax.ShapeDtypeStruct((N, C), jnp.float32),
            jax.ShapeDtypeStruct((N, C), jnp.float32),
        ],
    )


def kernel(x, W_enc, b_enc, W_dec, b_dec):
    B, C, H, W = x.shape
    D = W_enc.shape[0]
    N = B * H * W

    x_flat = jnp.transpose(x, (0, 2, 3, 1)).reshape(N, C)
    x_bf = x_flat.astype(jnp.bfloat16)
    we_t = W_enc.T.astype(jnp.bfloat16)
    wd_t = W_dec.T.astype(jnp.bfloat16)
    be = b_enc.reshape(1, D)
    bd = b_dec.reshape(1, C)

    br = 128 if N % 128 == 0 else N
    args = (x_bf, we_t, wd_t, be, bd)

    f16, f32_, r16, r32, flags = _make_fast_call(br, C, D, N)(*args)

    def _rare(_):
        return tuple(_make_exact_call(br, C, D, N)(*args))

    f16, f32_, r16, r32 = lax.cond(
        jnp.max(flags) > 0.0, _rare,
        lambda _: (f16, f32_, r16, r32), operand=None)

    recon16 = jnp.transpose(r16.reshape(B, H, W, C), (0, 3, 1, 2))
    recon32 = jnp.transpose(r32.reshape(B, H, W, C), (0, 3, 1, 2))
    return (f16, f32_, recon16, recon32)
